# baseline (speedup 1.0000x reference)
import functools

import jax
import jax.numpy as jnp
from jax import lax
from jax.experimental import pallas as pl
from jax.experimental.pallas import tpu as pltpu
from jax.experimental.pallas import tpu_sc as plsc

N = 10000
D = 128
NC = 2
NS = 16
NW = NC * NS
K = 96
NPAD = 10048
NPD = 10240
BR = 1000


def _pre_body(h_ref, wl_ref, wr_ref, att_ref, xl_ref, xr_ref, lref_ref):
    h = h_ref[...]
    xl = jnp.dot(h, wl_ref[...], preferred_element_type=jnp.float32)
    xr = jnp.dot(h, wr_ref[...], preferred_element_type=jnp.float32)
    xl_ref[...] = xl
    xr_ref[...] = xr
    u = xl + xr
    lu = jnp.where(u > 0, u, 0.2 * u)
    lref_ref[...] = jnp.dot(lu, att_ref[...], preferred_element_type=jnp.float32)


def _tc_pre(h, W_l, W_r, att):
    grid = N // BR
    return pl.pallas_call(
        _pre_body,
        grid=(grid,),
        in_specs=[
            pl.BlockSpec((BR, D), lambda i: (i, 0)),
            pl.BlockSpec((D, D), lambda i: (0, 0)),
            pl.BlockSpec((D, D), lambda i: (0, 0)),
            pl.BlockSpec((D, 1), lambda i: (0, 0)),
        ],
        out_specs=[
            pl.BlockSpec((BR, D), lambda i: (i, 0)),
            pl.BlockSpec((BR, D), lambda i: (i, 0)),
            pl.BlockSpec((BR, 1), lambda i: (i, 0)),
        ],
        out_shape=[
            jax.ShapeDtypeStruct((N, D), jnp.float32),
            jax.ShapeDtypeStruct((N, D), jnp.float32),
            jax.ShapeDtypeStruct((N, 1), jnp.float32),
        ],
    )(h, W_l, W_r, att.reshape(D, 1))


def _densum_body(den_ref, out_ref):
    out_ref[...] = jnp.sum(den_ref[...], axis=0, keepdims=True)


def _tc_densum(den):
    grid = NPD // D
    return pl.pallas_call(
        _densum_body,
        grid=(grid,),
        in_specs=[pl.BlockSpec((NW, D), lambda i: (0, i))],
        out_specs=pl.BlockSpec((1, D), lambda i: (0, i)),
        out_shape=jax.ShapeDtypeStruct((1, NPD), jnp.float32),
    )(den)


def _post_body(acc_ref, den_ref, lref_ref, xl_ref, b_ref, h_ref, out_ref):
    el = jnp.exp(lref_ref[...])
    num = acc_ref[0] + acc_ref[1] + el * xl_ref[...]
    den = den_ref[...] + el
    out = num / (den + 1e-16) + b_ref[...]
    out_ref[...] = jnp.maximum(out, 0.0) + h_ref[...]


def _postpre_body(acc_ref, den_ref, lrefp_ref, xl_ref, b_ref, h_ref,
                  wl_ref, wr_ref, att_ref,
                  out_ref, xl2_ref, xr2_ref, lref_ref):
    el = jnp.exp(lrefp_ref[...])
    num = acc_ref[0] + acc_ref[1] + el * xl_ref[...]
    den = den_ref[...] + el
    out = num / (den + 1e-16) + b_ref[...]
    h = jnp.maximum(out, 0.0) + h_ref[...]
    out_ref[...] = h
    xl = jnp.dot(h, wl_ref[...], preferred_element_type=jnp.float32)
    xr = jnp.dot(h, wr_ref[...], preferred_element_type=jnp.float32)
    xl2_ref[...] = xl
    xr2_ref[...] = xr
    u = xl + xr
    lu = jnp.where(u > 0, u, 0.2 * u)
    lref_ref[...] = jnp.dot(lu, att_ref[...], preferred_element_type=jnp.float32)


def _tc_postpre(acc, den, lref, xl, b, h, W_l, W_r, att):
    grid = N // BR
    return pl.pallas_call(
        _postpre_body,
        grid=(grid,),
        in_specs=[
            pl.BlockSpec((NC, BR, D), lambda i: (0, i, 0)),
            pl.BlockSpec((BR, 1), lambda i: (i, 0)),
            pl.BlockSpec((BR, 1), lambda i: (i, 0)),
            pl.BlockSpec((BR, D), lambda i: (i, 0)),
            pl.BlockSpec((1, D), lambda i: (0, 0)),
            pl.BlockSpec((BR, D), lambda i: (i, 0)),
            pl.BlockSpec((D, D), lambda i: (0, 0)),
            pl.BlockSpec((D, D), lambda i: (0, 0)),
            pl.BlockSpec((D, 1), lambda i: (0, 0)),
        ],
        out_specs=[
            pl.BlockSpec((BR, D), lambda i: (i, 0)),
            pl.BlockSpec((BR, D), lambda i: (i, 0)),
            pl.BlockSpec((BR, D), lambda i: (i, 0)),
            pl.BlockSpec((BR, 1), lambda i: (i, 0)),
        ],
        out_shape=[
            jax.ShapeDtypeStruct((N, D), jnp.float32),
            jax.ShapeDtypeStruct((N, D), jnp.float32),
            jax.ShapeDtypeStruct((N, D), jnp.float32),
            jax.ShapeDtypeStruct((N, 1), jnp.float32),
        ],
    )(acc, den, lref, xl, b.reshape(1, D), h, W_l, W_r, att.reshape(D, 1))


def _tc_post(acc, den, lref, xl, b, h):
    grid = N // BR
    return pl.pallas_call(
        _post_body,
        grid=(grid,),
        in_specs=[
            pl.BlockSpec((NC, BR, D), lambda i: (0, i, 0)),
            pl.BlockSpec((BR, 1), lambda i: (i, 0)),
            pl.BlockSpec((BR, 1), lambda i: (i, 0)),
            pl.BlockSpec((BR, D), lambda i: (i, 0)),
            pl.BlockSpec((1, D), lambda i: (0, 0)),
            pl.BlockSpec((BR, D), lambda i: (i, 0)),
        ],
        out_specs=pl.BlockSpec((BR, D), lambda i: (i, 0)),
        out_shape=jax.ShapeDtypeStruct((N, D), jnp.float32),
    )(acc, den, lref, xl, b.reshape(1, D), h)


def _sc_body(E, CPT, xl_hbm, xr_hbm, att_hbm, src_hbm, dst_hbm,
             acc_hbm, den_hbm,
             att_v, den_v, accb_v,
             src_v0, src_v1, dst_v0, dst_v1, dsc_v0, dsc_v1, xlr_v0, xlr_v1,
             xrr_v, acc_sh,
             sem_ix0, sem_ix1, sem_xl0, sem_xl1, sem_xr0,
             sem_sc0, sem_sc1):
    c = lax.axis_index("c")
    s = lax.axis_index("s")
    wid = c * NS + s

    srcs = [src_v0, src_v1]
    dstv = [dst_v0, dst_v1]
    dscv = [dsc_v0, dsc_v1]
    xls = [xlr_v0, xlr_v1]
    sem_ix = [sem_ix0, sem_ix1]
    sem_xl = [sem_xl0, sem_xl1]
    sem_sc = [sem_sc0, sem_sc1]

    pltpu.sync_copy(att_hbm, att_v)

    zero16 = jnp.zeros((16,), jnp.float32)

    def _zden(i, _):
        den_v[pl.ds(i * 16, 16)] = zero16
        return 0
    lax.fori_loop(0, NPD // 16, _zden, 0)

    def _zrow(i, _):
        for dv in range(D // 16):
            xlr_v0[i, pl.ds(dv * 16, 16)] = zero16
        return 0
    lax.fori_loop(0, K, _zrow, 0)
    nch = NPAD // 64

    def _zacc(i, _):
        ch = i * NS + s

        @pl.when(ch < nch)
        def _():
            pltpu.sync_copy(xlr_v0.at[pl.ds(0, 64)],
                            acc_sh.at[pl.ds(ch * 64, 64)])
        return 0
    lax.fori_loop(0, -(-nch // NS), _zacc, 0)
    plsc.subcore_barrier()

    att8 = [att_v[pl.ds(dv * 16, 16)] for dv in range(8)]
    rows16 = lax.iota(jnp.int32, 16)
    tbase = wid * CPT * K

    def issue_idx(j, b):
        base = tbase + j * K
        pltpu.async_copy(src_hbm.at[pl.ds(base, K)], srcs[b], sem_ix[b])
        pltpu.async_copy(dst_hbm.at[pl.ds(base, K)], dstv[b], sem_ix[b])

    def wait_idx(b):
        pltpu.make_async_copy(src_hbm.at[pl.ds(0, K)], srcs[b],
                              sem_ix[b]).wait()
        pltpu.make_async_copy(dst_hbm.at[pl.ds(0, K)], dstv[b],
                              sem_ix[b]).wait()

    def issue_xl(b):
        pltpu.async_copy(xl_hbm.at[srcs[b]], xls[b], sem_xl[b])

    def wait_xl(b):
        pltpu.make_async_copy(xl_hbm.at[srcs[b]], xls[b], sem_xl[b]).wait()

    def issue_xr(b):
        pltpu.async_copy(xr_hbm.at[dstv[b]], xrr_v, sem_xr0)

    def wait_xr(b):
        pltpu.make_async_copy(xr_hbm.at[dstv[b]], xrr_v, sem_xr0).wait()

    def issue_scatter(b):
        for q in range(K // 16):
            dscv[b][pl.ds(q * 16, 16)] = dstv[b][pl.ds(q * 16, 16)]
        pltpu.async_copy(xls[b], acc_sh.at[dscv[b]], sem_sc[b], add=True)

    def wait_scatter(b):
        pltpu.make_async_copy(xls[b], acc_sh.at[dscv[b]], sem_sc[b]).wait()

    def compute(j, b):
        xlr_v = xls[b]
        base = tbase + j * K

        def _group(g, _):
            off = g * 16
            for ee in range(16):
                e = off + ee
                acc = zero16
                for dv in range(8):
                    a = xlr_v[e, pl.ds(dv * 16, 16)]
                    b2 = xrr_v[e, pl.ds(dv * 16, 16)]
                    u = a + b2
                    lu = jnp.where(u > 0, u, 0.2 * u)
                    acc = acc + lu * att8[dv]
                accb_v[pl.ds(ee * 16, 16)] = acc
            logit = zero16
            rows16x = rows16 * 16
            for k in range(16):
                colk = plsc.load_gather(accb_v, [rows16x + k])
                logit = logit + colk
            dsts = dstv[b][pl.ds(off, 16)]
            p = jnp.exp(logit)
            valid = (base + off + rows16) < E
            p = jnp.where(valid, p, 0.0)
            plsc.addupdate_scatter(den_v, [dsts], p)
            for ee in range(16):
                e = off + ee
                pe = p[ee]
                for dv in range(8):
                    xlr_v[e, pl.ds(dv * 16, 16)] = (
                        xlr_v[e, pl.ds(dv * 16, 16)] * pe)
            return 0

        lax.fori_loop(0, K // 16, _group, 0)

    issue_idx(0, 0)
    wait_idx(0)
    issue_xl(0)
    issue_xr(0)
    issue_idx(1, 1)

    def _iter(i, _):
        for b in range(2):
            j = i * 2 + b
            nb = 1 - b
            wait_xl(b)
            wait_xr(b)

            @pl.when(j >= 1)
            def _():
                wait_scatter(nb)

            @pl.when(j + 1 < CPT)
            def _():
                wait_idx(nb)
                issue_xl(nb)

            compute(j, b)
            issue_scatter(b)

            @pl.when(j + 1 < CPT)
            def _():
                issue_xr(nb)

            @pl.when(j + 2 < CPT)
            def _():
                issue_idx(j + 2, b)
        return 0

    lax.fori_loop(0, CPT // 2, _iter, 0)
    wait_scatter(1)
    pltpu.sync_copy(den_v, den_hbm.at[wid])
    plsc.subcore_barrier()

    @pl.when(s == 0)
    def _():
        pltpu.sync_copy(acc_sh.at[pl.ds(0, N)], acc_hbm.at[c])


def _sc_edge(E, Epad, xl, xr, att, srcp, dstp):
    CPT = Epad // (NW * K)
    mesh = plsc.VectorSubcoreMesh(core_axis_name="c", subcore_axis_name="s")
    fn = pl.kernel(
        functools.partial(_sc_body, E, CPT),
        out_type=[
            jax.ShapeDtypeStruct((NC, N, D), jnp.float32),
            jax.ShapeDtypeStruct((NW, NPD), jnp.float32),
        ],
        mesh=mesh,
        compiler_params=pltpu.CompilerParams(needs_layout_passes=False),
        scratch_types=[
            pltpu.VMEM((D,), jnp.float32),
            pltpu.VMEM((NPD,), jnp.float32),
            pltpu.VMEM((256,), jnp.float32),
            pltpu.VMEM((K,), jnp.int32),
            pltpu.VMEM((K,), jnp.int32),
            pltpu.VMEM((K,), jnp.int32),
            pltpu.VMEM((K,), jnp.int32),
            pltpu.VMEM((K,), jnp.int32),
            pltpu.VMEM((K,), jnp.int32),
            pltpu.VMEM((K, D), jnp.float32),
            pltpu.VMEM((K, D), jnp.float32),
            pltpu.VMEM((K, D), jnp.float32),
            pltpu.VMEM_SHARED((NPAD, D), jnp.float32),
        ] + [pltpu.SemaphoreType.DMA] * 7,
    )
    return fn(xl, xr, att, srcp, dstp)


def kernel(x, edge_index, W_l0, W_r0, att0, b0, W_l1, W_r1, att1, b1,
           W_l2, W_r2, att2, b2):
    E = edge_index.shape[1]
    cpt = -(-E // (NW * K))
    cpt = cpt + (cpt % 2)
    Epad = NW * K * cpt
    pad = Epad - E
    srcp = jnp.concatenate([edge_index[0], jnp.zeros((pad,), jnp.int32)])
    dstp = jnp.concatenate([edge_index[1], jnp.zeros((pad,), jnp.int32)])

    params = [(W_l0, W_r0, att0, b0), (W_l1, W_r1, att1, b1),
              (W_l2, W_r2, att2, b2)]
    h = x
    xl, xr, lref = _tc_pre(h, params[0][0], params[0][1], params[0][2])
    for l, (W_l, W_r, att, b) in enumerate(params):
        acc, den = _sc_edge(E, Epad, xl, xr, att, srcp, dstp)
        den1 = _tc_densum(den).reshape(NPD)[:N].reshape(N, 1)
        if l + 1 < len(params):
            nW_l, nW_r, natt, _ = params[l + 1]
            h, xl, xr, lref = _tc_postpre(acc, den1, lref, xl, b, h,
                                          nW_l, nW_r, natt)
        else:
            h = _tc_post(acc, den1, lref, xl, b, h)
    return h


# speedup vs baseline: 1.3549x; 1.3549x over previous
import functools

import jax
import jax.numpy as jnp
from jax import lax
from jax.experimental import pallas as pl
from jax.experimental.pallas import tpu as pltpu
from jax.experimental.pallas import tpu_sc as plsc

N = 10000
D = 128
NC = 2
NS = 16
NW = NC * NS
K = 64
NPAD = 10048
NPD = 10240
BR = 1000


def _pre_body(h_ref, wl_ref, wr_ref, att_ref, xl_ref, xr_ref, lref_ref):
    h = h_ref[...]
    xl = jnp.dot(h, wl_ref[...], preferred_element_type=jnp.float32)
    xr = jnp.dot(h, wr_ref[...], preferred_element_type=jnp.float32)
    xl_ref[...] = xl
    xr_ref[...] = xr
    u = xl + xr
    lu = jnp.where(u > 0, u, 0.2 * u)
    lref_ref[...] = jnp.dot(lu, att_ref[...], preferred_element_type=jnp.float32)


def _tc_pre(h, W_l, W_r, att):
    grid = N // BR
    return pl.pallas_call(
        _pre_body,
        grid=(grid,),
        in_specs=[
            pl.BlockSpec((BR, D), lambda i: (i, 0)),
            pl.BlockSpec((D, D), lambda i: (0, 0)),
            pl.BlockSpec((D, D), lambda i: (0, 0)),
            pl.BlockSpec((D, 1), lambda i: (0, 0)),
        ],
        out_specs=[
            pl.BlockSpec((BR, D), lambda i: (i, 0)),
            pl.BlockSpec((BR, D), lambda i: (i, 0)),
            pl.BlockSpec((BR, 1), lambda i: (i, 0)),
        ],
        out_shape=[
            jax.ShapeDtypeStruct((N, D), jnp.float32),
            jax.ShapeDtypeStruct((N, D), jnp.float32),
            jax.ShapeDtypeStruct((N, 1), jnp.float32),
        ],
    )(h, W_l, W_r, att.reshape(D, 1))


def _densum_body(den_ref, out_ref):
    out_ref[...] = jnp.sum(den_ref[...], axis=0, keepdims=True)


def _tc_densum(den):
    grid = NPD // D
    return pl.pallas_call(
        _densum_body,
        grid=(grid,),
        in_specs=[pl.BlockSpec((NW, D), lambda i: (0, i))],
        out_specs=pl.BlockSpec((1, D), lambda i: (0, i)),
        out_shape=jax.ShapeDtypeStruct((1, NPD), jnp.float32),
    )(den)


def _post_body(acc_ref, den_ref, lref_ref, xl_ref, b_ref, h_ref, out_ref):
    el = jnp.exp(lref_ref[...])
    num = acc_ref[0] + acc_ref[1] + el * xl_ref[...]
    den = den_ref[...] + el
    out = num / (den + 1e-16) + b_ref[...]
    out_ref[...] = jnp.maximum(out, 0.0) + h_ref[...]


def _postpre_body(acc_ref, den_ref, lrefp_ref, xl_ref, b_ref, h_ref,
                  wl_ref, wr_ref, att_ref,
                  out_ref, xl2_ref, xr2_ref, lref_ref):
    el = jnp.exp(lrefp_ref[...])
    num = acc_ref[0] + acc_ref[1] + el * xl_ref[...]
    den = den_ref[...] + el
    out = num / (den + 1e-16) + b_ref[...]
    h = jnp.maximum(out, 0.0) + h_ref[...]
    out_ref[...] = h
    xl = jnp.dot(h, wl_ref[...], preferred_element_type=jnp.float32)
    xr = jnp.dot(h, wr_ref[...], preferred_element_type=jnp.float32)
    xl2_ref[...] = xl
    xr2_ref[...] = xr
    u = xl + xr
    lu = jnp.where(u > 0, u, 0.2 * u)
    lref_ref[...] = jnp.dot(lu, att_ref[...], preferred_element_type=jnp.float32)


def _tc_postpre(acc, den, lref, xl, b, h, W_l, W_r, att):
    grid = N // BR
    return pl.pallas_call(
        _postpre_body,
        grid=(grid,),
        in_specs=[
            pl.BlockSpec((NC, BR, D), lambda i: (0, i, 0)),
            pl.BlockSpec((BR, 1), lambda i: (i, 0)),
            pl.BlockSpec((BR, 1), lambda i: (i, 0)),
            pl.BlockSpec((BR, D), lambda i: (i, 0)),
            pl.BlockSpec((1, D), lambda i: (0, 0)),
            pl.BlockSpec((BR, D), lambda i: (i, 0)),
            pl.BlockSpec((D, D), lambda i: (0, 0)),
            pl.BlockSpec((D, D), lambda i: (0, 0)),
            pl.BlockSpec((D, 1), lambda i: (0, 0)),
        ],
        out_specs=[
            pl.BlockSpec((BR, D), lambda i: (i, 0)),
            pl.BlockSpec((BR, D), lambda i: (i, 0)),
            pl.BlockSpec((BR, D), lambda i: (i, 0)),
            pl.BlockSpec((BR, 1), lambda i: (i, 0)),
        ],
        out_shape=[
            jax.ShapeDtypeStruct((N, D), jnp.float32),
            jax.ShapeDtypeStruct((N, D), jnp.float32),
            jax.ShapeDtypeStruct((N, D), jnp.float32),
            jax.ShapeDtypeStruct((N, 1), jnp.float32),
        ],
    )(acc, den, lref, xl, b.reshape(1, D), h, W_l, W_r, att.reshape(D, 1))


def _tc_post(acc, den, lref, xl, b, h):
    grid = N // BR
    return pl.pallas_call(
        _post_body,
        grid=(grid,),
        in_specs=[
            pl.BlockSpec((NC, BR, D), lambda i: (0, i, 0)),
            pl.BlockSpec((BR, 1), lambda i: (i, 0)),
            pl.BlockSpec((BR, 1), lambda i: (i, 0)),
            pl.BlockSpec((BR, D), lambda i: (i, 0)),
            pl.BlockSpec((1, D), lambda i: (0, 0)),
            pl.BlockSpec((BR, D), lambda i: (i, 0)),
        ],
        out_specs=pl.BlockSpec((BR, D), lambda i: (i, 0)),
        out_shape=jax.ShapeDtypeStruct((N, D), jnp.float32),
    )(acc, den, lref, xl, b.reshape(1, D), h)


def _sc_body(E, CPT, xl_hbm, xr_hbm, att_hbm, src_hbm, dst_hbm,
             acc_hbm, den_hbm,
             att_v, den_v, accb_v,
             src_v0, src_v1, dst_v0, dst_v1, dsc_v0, dsc_v1, xlr_v0, xlr_v1,
             xrr_v0, xrr_v1, acc_sh,
             sem_ix0, sem_ix1, sem_xl0, sem_xl1, sem_xr0, sem_xr1,
             sem_sc0, sem_sc1):
    c = lax.axis_index("c")
    s = lax.axis_index("s")
    wid = c * NS + s

    srcs = [src_v0, src_v1]
    dstv = [dst_v0, dst_v1]
    dscv = [dsc_v0, dsc_v1]
    xls = [xlr_v0, xlr_v1]
    xrs = [xrr_v0, xrr_v1]
    sem_ix = [sem_ix0, sem_ix1]
    sem_xl = [sem_xl0, sem_xl1]
    sem_xr = [sem_xr0, sem_xr1]
    sem_sc = [sem_sc0, sem_sc1]

    pltpu.sync_copy(att_hbm, att_v)

    zero16 = jnp.zeros((16,), jnp.float32)

    pltpu.async_copy(src_hbm.at[pl.ds(wid * CPT * K, K)], src_v0, sem_ix0)
    pltpu.async_copy(dst_hbm.at[pl.ds(wid * CPT * K, K)], dst_v0, sem_ix0)
    pltpu.async_copy(src_hbm.at[pl.ds((wid * CPT + 1) * K, K)], src_v1,
                     sem_ix1)
    pltpu.async_copy(dst_hbm.at[pl.ds((wid * CPT + 1) * K, K)], dst_v1,
                     sem_ix1)

    def _zden(i, _):
        den_v[pl.ds(i * 16, 16)] = zero16
        return 0
    lax.fori_loop(0, NPD // 16, _zden, 0)

    def _zrow(i, _):
        for dv in range(D // 16):
            xlr_v0[i, pl.ds(dv * 16, 16)] = zero16
        return 0
    lax.fori_loop(0, K, _zrow, 0)
    nch = NPAD // 64

    def _zacc(i, _):
        ch = i * NS + s

        @pl.when(ch < nch)
        def _():
            pltpu.sync_copy(xlr_v0.at[pl.ds(0, 64)],
                            acc_sh.at[pl.ds(ch * 64, 64)])
        return 0
    lax.fori_loop(0, -(-nch // NS), _zacc, 0)

    att8 = [att_v[pl.ds(dv * 16, 16)] for dv in range(8)]
    rows16 = lax.iota(jnp.int32, 16)
    tbase = wid * CPT * K

    def issue_idx(j, b):
        base = tbase + j * K
        pltpu.async_copy(src_hbm.at[pl.ds(base, K)], srcs[b], sem_ix[b])
        pltpu.async_copy(dst_hbm.at[pl.ds(base, K)], dstv[b], sem_ix[b])

    def wait_idx(b):
        pltpu.make_async_copy(src_hbm.at[pl.ds(0, K)], srcs[b],
                              sem_ix[b]).wait()
        pltpu.make_async_copy(dst_hbm.at[pl.ds(0, K)], dstv[b],
                              sem_ix[b]).wait()

    def issue_gather(b):
        pltpu.async_copy(xl_hbm.at[srcs[b]], xls[b], sem_xl[b])
        pltpu.async_copy(xr_hbm.at[dstv[b]], xrs[b], sem_xr[b])

    def wait_gather(b):
        pltpu.make_async_copy(xl_hbm.at[srcs[b]], xls[b], sem_xl[b]).wait()
        pltpu.make_async_copy(xr_hbm.at[dstv[b]], xrs[b], sem_xr[b]).wait()

    def issue_scatter(b):
        for q in range(K // 16):
            dscv[b][pl.ds(q * 16, 16)] = dstv[b][pl.ds(q * 16, 16)]
        pltpu.async_copy(xls[b], acc_sh.at[dscv[b]], sem_sc[b], add=True)

    def wait_scatter(b):
        pltpu.make_async_copy(xls[b], acc_sh.at[dscv[b]], sem_sc[b]).wait()

    def compute(j, b):
        xlr_v = xls[b]
        xrr_v = xrs[b]
        base = tbase + j * K

        def _group(g, _):
            off = g * 16
            for ee in range(16):
                e = off + ee
                acc = zero16
                for dv in range(8):
                    a = xlr_v[e, pl.ds(dv * 16, 16)]
                    b2 = xrr_v[e, pl.ds(dv * 16, 16)]
                    u = a + b2
                    lu = jnp.where(u > 0, u, 0.2 * u)
                    acc = acc + lu * att8[dv]
                accb_v[pl.ds(ee * 16, 16)] = acc
            logit = zero16
            rows16x = rows16 * 16
            for k in range(16):
                colk = plsc.load_gather(accb_v, [rows16x + k])
                logit = logit + colk
            dsts = dstv[b][pl.ds(off, 16)]
            p = jnp.exp(logit)
            valid = (base + off + rows16) < E
            p = jnp.where(valid, p, 0.0)
            plsc.addupdate_scatter(den_v, [dsts], p)
            for ee in range(16):
                e = off + ee
                pe = p[ee]
                for dv in range(8):
                    xlr_v[e, pl.ds(dv * 16, 16)] = (
                        xlr_v[e, pl.ds(dv * 16, 16)] * pe)
            return 0

        lax.fori_loop(0, K // 16, _group, 0)

    wait_idx(0)
    issue_gather(0)
    plsc.subcore_barrier()

    def _iter(i, _):
        for b in range(2):
            j = i * 2 + b
            nb = 1 - b
            wait_gather(b)

            @pl.when(j >= 1)
            def _():
                wait_scatter(nb)

            @pl.when(j + 1 < CPT)
            def _():
                wait_idx(nb)
                issue_gather(nb)

            compute(j, b)
            issue_scatter(b)

            @pl.when(j + 2 < CPT)
            def _():
                issue_idx(j + 2, b)
        return 0

    lax.fori_loop(0, CPT // 2, _iter, 0)
    wait_scatter(1)
    pltpu.sync_copy(den_v, den_hbm.at[wid])
    plsc.subcore_barrier()

    @pl.when(s == 0)
    def _():
        pltpu.sync_copy(acc_sh.at[pl.ds(0, N)], acc_hbm.at[c])


def _sc_edge(E, Epad, xl, xr, att, srcp, dstp):
    CPT = Epad // (NW * K)
    mesh = plsc.VectorSubcoreMesh(core_axis_name="c", subcore_axis_name="s")
    fn = pl.kernel(
        functools.partial(_sc_body, E, CPT),
        out_type=[
            jax.ShapeDtypeStruct((NC, N, D), jnp.float32),
            jax.ShapeDtypeStruct((NW, NPD), jnp.float32),
        ],
        mesh=mesh,
        compiler_params=pltpu.CompilerParams(needs_layout_passes=False),
        scratch_types=[
            pltpu.VMEM((D,), jnp.float32),
            pltpu.VMEM((NPD,), jnp.float32),
            pltpu.VMEM((256,), jnp.float32),
            pltpu.VMEM((K,), jnp.int32),
            pltpu.VMEM((K,), jnp.int32),
            pltpu.VMEM((K,), jnp.int32),
            pltpu.VMEM((K,), jnp.int32),
            pltpu.VMEM((K,), jnp.int32),
            pltpu.VMEM((K,), jnp.int32),
            pltpu.VMEM((K, D), jnp.float32),
            pltpu.VMEM((K, D), jnp.float32),
            pltpu.VMEM((K, D), jnp.float32),
            pltpu.VMEM((K, D), jnp.float32),
            pltpu.VMEM_SHARED((NPAD, D), jnp.float32),
        ] + [pltpu.SemaphoreType.DMA] * 8,
    )
    return fn(xl, xr, att, srcp, dstp)


def kernel(x, edge_index, W_l0, W_r0, att0, b0, W_l1, W_r1, att1, b1,
           W_l2, W_r2, att2, b2):
    E = edge_index.shape[1]
    cpt = -(-E // (NW * K))
    cpt = cpt + (cpt % 2)
    Epad = NW * K * cpt
    pad = Epad - E
    srcp = jnp.concatenate([edge_index[0], jnp.zeros((pad,), jnp.int32)])
    dstp = jnp.concatenate([edge_index[1], jnp.zeros((pad,), jnp.int32)])

    params = [(W_l0, W_r0, att0, b0), (W_l1, W_r1, att1, b1),
              (W_l2, W_r2, att2, b2)]
    h = x
    xl, xr, lref = _tc_pre(h, params[0][0], params[0][1], params[0][2])
    for l, (W_l, W_r, att, b) in enumerate(params):
        acc, den = _sc_edge(E, Epad, xl, xr, att, srcp, dstp)
        den1 = _tc_densum(den).reshape(NPD)[:N].reshape(N, 1)
        if l + 1 < len(params):
            nW_l, nW_r, natt, _ = params[l + 1]
            h, xl, xr, lref = _tc_postpre(acc, den1, lref, xl, b, h,
                                          nW_l, nW_r, natt)
        else:
            h = _tc_post(acc, den1, lref, xl, b, h)
    return h
